# Initial kernel scaffold; baseline (speedup 1.0000x reference)
import jax
import jax.numpy as jnp
from jax.experimental import pallas as pl


def kernel(x, gate, w1, w2, w3):
    raise NotImplementedError("write your pallas kernel here")


# speedup vs baseline: 1.8631x; 1.8631x over previous
import functools

import jax
import jax.numpy as jnp
from jax.experimental import pallas as pl

T = 4096
D = 1024
H = 512
E = 16
K = 2

BT = 2048


def _moe_dense_body(x_ref, gt_ref, w1_ref, w3_ref, w2_ref, out_ref):
    e = pl.program_id(1)
    x = x_ref[...]
    logits = jnp.dot(x, gt_ref[...], preferred_element_type=jnp.float32)
    m = jnp.max(logits, axis=-1, keepdims=True)
    ex = jnp.exp(logits - m)
    scores = ex / jnp.sum(ex, axis=-1, keepdims=True)

    iota = jax.lax.broadcasted_iota(jnp.int32, logits.shape, 1)
    m1 = jnp.max(scores, axis=-1, keepdims=True)
    i1 = jnp.min(jnp.where(scores == m1, iota, E), axis=-1, keepdims=True)
    masked = jnp.where(iota == i1, -jnp.inf, scores)
    m2 = jnp.max(masked, axis=-1, keepdims=True)
    i2 = jnp.min(jnp.where(masked == m2, iota, E), axis=-1, keepdims=True)
    w_e = jnp.where(i1 == e, m1, 0.0) + jnp.where(i2 == e, m2, 0.0)

    w1 = w1_ref[0]
    w3 = w3_ref[0]
    w2 = w2_ref[0]
    a = jax.lax.dot_general(x, w1, (((1,), (1,)), ((), ())),
                            preferred_element_type=jnp.float32)
    b = jax.lax.dot_general(x, w3, (((1,), (1,)), ((), ())),
                            preferred_element_type=jnp.float32)
    h = (a * jax.nn.sigmoid(a)) * b
    y = jax.lax.dot_general(h, w2, (((1,), (1,)), ((), ())),
                            preferred_element_type=jnp.float32)

    @pl.when(e == 0)
    def _():
        out_ref[...] = jnp.zeros_like(out_ref)

    out_ref[...] += y * w_e


@functools.partial(jax.jit, static_argnames=("interpret",))
def kernel(x, gate, w1, w2, w3, interpret=False):
    gate_t = gate.T
    out = pl.pallas_call(
        _moe_dense_body,
        grid=(T // BT, E),
        in_specs=[
            pl.BlockSpec((BT, D), lambda i, e: (i, 0)),
            pl.BlockSpec((D, E), lambda i, e: (0, 0)),
            pl.BlockSpec((1, H, D), lambda i, e: (e, 0, 0)),
            pl.BlockSpec((1, H, D), lambda i, e: (e, 0, 0)),
            pl.BlockSpec((1, D, H), lambda i, e: (e, 0, 0)),
        ],
        out_specs=pl.BlockSpec((BT, D), lambda i, e: (i, 0)),
        out_shape=jax.ShapeDtypeStruct((T, D), jnp.float32),
        interpret=interpret,
    )(x, gate_t, w1, w3, w2)
    return out


# speedup vs baseline: 3.2396x; 1.7388x over previous
import functools

import jax
import jax.numpy as jnp
from jax import lax
from jax.experimental import pallas as pl
from jax.experimental.pallas import tpu as pltpu
from jax.experimental.pallas import tpu_sc as plsc

T = 4096
D = 1024
H = 512
E = 16
K = 2

BLK = 256
P = T * K + E * BLK
NB = P // BLK

NC = 2
NS = 16
NW = NC * NS
TW = T // NW
CHD = 64
CHC = 32


def _router_body(x_ref, gt_ref, pos_ref, wts_ref, be_ref):
    x = x_ref[...]
    logits = jnp.dot(x, gt_ref[...], preferred_element_type=jnp.float32)
    m = jnp.max(logits, axis=-1, keepdims=True)
    ex = jnp.exp(logits - m)
    scores = ex / jnp.sum(ex, axis=-1, keepdims=True)

    eiota = lax.broadcasted_iota(jnp.int32, (T, E), 1)
    m1 = jnp.max(scores, axis=-1, keepdims=True)
    i1 = jnp.min(jnp.where(scores == m1, eiota, E), axis=-1, keepdims=True)
    masked = jnp.where(eiota == i1, -jnp.inf, scores)
    m2 = jnp.max(masked, axis=-1, keepdims=True)
    i2 = jnp.min(jnp.where(masked == m2, eiota, E), axis=-1, keepdims=True)

    oh1 = (i1 == eiota).astype(jnp.float32)
    oh2 = (i2 == eiota).astype(jnp.float32)
    cnt1 = jnp.sum(oh1, axis=0, keepdims=True)
    cnt2 = jnp.sum(oh2, axis=0, keepdims=True)
    cnt = cnt1 + cnt2

    pc = jnp.ceil(cnt * (1.0 / BLK)) * BLK
    li = lax.broadcasted_iota(jnp.int32, (E, E), 0)
    lj = lax.broadcasted_iota(jnp.int32, (E, E), 1)
    lmat = (li < lj).astype(jnp.float32)
    starts = jnp.dot(pc, lmat, preferred_element_type=jnp.float32)
    ends = starts + pc

    C = 512
    ri = lax.broadcasted_iota(jnp.int32, (C, C), 0)
    rj = lax.broadcasted_iota(jnp.int32, (C, C), 1)
    tri = (rj < ri).astype(jnp.float32)
    carry1 = jnp.zeros((1, E), jnp.float32)
    carry2 = cnt1
    r1p, r2p = [], []
    for c in range(T // C):
        b1 = oh1[c * C:(c + 1) * C]
        b2 = oh2[c * C:(c + 1) * C]
        e1 = jnp.dot(tri, b1, preferred_element_type=jnp.float32) + carry1
        e2 = jnp.dot(tri, b2, preferred_element_type=jnp.float32) + carry2
        r1p.append(jnp.sum(b1 * e1, axis=1, keepdims=True))
        r2p.append(jnp.sum(b2 * e2, axis=1, keepdims=True))
        carry1 = carry1 + jnp.sum(b1, axis=0, keepdims=True)
        carry2 = carry2 + jnp.sum(b2, axis=0, keepdims=True)
    r1 = jnp.concatenate(r1p, axis=0)
    r2 = jnp.concatenate(r2p, axis=0)

    s1 = jnp.sum(oh1 * starts, axis=1, keepdims=True)
    s2 = jnp.sum(oh2 * starts, axis=1, keepdims=True)
    p1 = (s1 + r1).astype(jnp.int32)
    p2 = (s2 + r2).astype(jnp.int32)

    pos_ref[...] = jnp.concatenate([p1, p2], axis=1)
    wts_ref[...] = jnp.concatenate(
        [jnp.broadcast_to(m1, (T, 16)), jnp.broadcast_to(m2, (T, 16))], axis=1)

    bstart = (lax.broadcasted_iota(jnp.int32, (128, 1), 0) * BLK
              ).astype(jnp.float32)
    be_ref[...] = jnp.sum((ends <= bstart).astype(jnp.int32), axis=1,
                          keepdims=True)


def _router(x, gate_t):
    return pl.pallas_call(
        _router_body,
        grid=(1,),
        in_specs=[
            pl.BlockSpec((T, D), lambda i: (0, 0)),
            pl.BlockSpec((D, E), lambda i: (0, 0)),
        ],
        out_specs=[
            pl.BlockSpec((T, K), lambda i: (0, 0)),
            pl.BlockSpec((T, K * 16), lambda i: (0, 0)),
            pl.BlockSpec((128, 1), lambda i: (0, 0)),
        ],
        out_shape=[
            jax.ShapeDtypeStruct((T, K), jnp.int32),
            jax.ShapeDtypeStruct((T, K * 16), jnp.float32),
            jax.ShapeDtypeStruct((128, 1), jnp.int32),
        ],
    )(x, gate_t)


def _dispatch_body(x_hbm, posd_hbm, xs_hbm, idxb, xbuf, sem):
    wid = lax.axis_index("s") * NC + lax.axis_index("c")
    base = wid * TW
    for c in range(TW // CHD):
        pltpu.sync_copy(posd_hbm.at[wid, c], idxb)
        pltpu.sync_copy(x_hbm.at[pl.ds(base + c * CHD, CHD)], xbuf)
        pltpu.async_copy(xbuf, xs_hbm.at[idxb.at[0]], sem).wait()
        pltpu.async_copy(xbuf, xs_hbm.at[idxb.at[1]], sem).wait()


@functools.cache
def _sc_mesh():
    return plsc.VectorSubcoreMesh(core_axis_name="c", subcore_axis_name="s",
                                  num_cores=NC, num_subcores=NS)


@functools.cache
def _dispatch():
    return pl.kernel(
        _dispatch_body,
        out_type=jax.ShapeDtypeStruct((P, D), jnp.float32),
        mesh=_sc_mesh(),
        scratch_types=[
            pltpu.VMEM((K, CHD), jnp.int32),
            pltpu.VMEM((CHD, D), jnp.float32),
            pltpu.SemaphoreType.DMA,
        ],
    )


def _gmm_body(be_ref, xs_ref, w1_ref, w3_ref, w2_ref, ys_ref):
    b = pl.program_id(0)

    @pl.when(be_ref[b] < E)
    def _():
        x = xs_ref[...]
        a = lax.dot_general(x, w1_ref[0], (((1,), (1,)), ((), ())),
                            preferred_element_type=jnp.float32)
        g = lax.dot_general(x, w3_ref[0], (((1,), (1,)), ((), ())),
                            preferred_element_type=jnp.float32)
        h = (a * jax.nn.sigmoid(a)) * g
        ys_ref[...] = lax.dot_general(h, w2_ref[0], (((1,), (1,)), ((), ())),
                                      preferred_element_type=jnp.float32)


def _gmm(be_flat, xs, w1, w3, w2):
    def wsel(b, be):
        return (jnp.minimum(be[b], E - 1), 0, 0)

    grid_spec = pltpu.PrefetchScalarGridSpec(
        num_scalar_prefetch=1,
        grid=(NB,),
        in_specs=[
            pl.BlockSpec((BLK, D), lambda b, be: (b, 0)),
            pl.BlockSpec((1, H, D), wsel),
            pl.BlockSpec((1, H, D), wsel),
            pl.BlockSpec((1, D, H), wsel),
        ],
        out_specs=pl.BlockSpec((BLK, D), lambda b, be: (b, 0)),
    )
    return pl.pallas_call(
        _gmm_body,
        grid_spec=grid_spec,
        out_shape=jax.ShapeDtypeStruct((P, D), jnp.float32),
    )(be_flat, xs, w1, w3, w2)


def _combine_body(ys_hbm, posc_hbm, wtc_hbm, out_hbm, idxb, wb, y0, y1,
                  sem0, sem1):
    wid = lax.axis_index("s") * NC + lax.axis_index("c")
    base = wid * TW
    for c in range(TW // CHC):
        pltpu.sync_copy(posc_hbm.at[wid, c], idxb)
        pltpu.sync_copy(wtc_hbm.at[wid, c], wb)
        cp0 = pltpu.async_copy(ys_hbm.at[idxb.at[0]], y0, sem0)
        cp1 = pltpu.async_copy(ys_hbm.at[idxb.at[1]], y1, sem1)
        cp0.wait()
        cp1.wait()

        def tok(j, _):
            w0 = wb[j, pl.ds(0, 16)]
            w1v = wb[j, pl.ds(16, 16)]
            for v in range(D // 16):
                sl = pl.ds(v * 16, 16)
                y0[j, sl] = w0 * y0[j, sl] + w1v * y1[j, sl]
            return 0

        lax.fori_loop(0, CHC, tok, 0)
        pltpu.sync_copy(y0, out_hbm.at[pl.ds(base + c * CHC, CHC)])


@functools.cache
def _combine():
    return pl.kernel(
        _combine_body,
        out_type=jax.ShapeDtypeStruct((T, D), jnp.float32),
        mesh=_sc_mesh(),
        scratch_types=[
            pltpu.VMEM((K, CHC), jnp.int32),
            pltpu.VMEM((CHC, K * 16), jnp.float32),
            pltpu.VMEM((CHC, D), jnp.float32),
            pltpu.VMEM((CHC, D), jnp.float32),
            pltpu.SemaphoreType.DMA,
            pltpu.SemaphoreType.DMA,
        ],
    )


@jax.jit
def kernel(x, gate, w1, w2, w3):
    pos, wts, be = _router(x, gate.T)
    be_flat = be.reshape(128)

    posd = pos.reshape(NW, TW // CHD, CHD, K).transpose(0, 1, 3, 2)
    posc = pos.reshape(NW, TW // CHC, CHC, K).transpose(0, 1, 3, 2)
    wtc = wts.reshape(NW, TW // CHC, CHC, K * 16)

    xs = _dispatch()(x, posd)
    ys = _gmm(be_flat, xs, w1, w3, w2)
    return _combine()(ys, posc, wtc)
